# baseline (speedup 1.0000x reference)
import jax
import jax.numpy as jnp
from jax import lax
from jax.experimental import pallas as pl
from jax.experimental.pallas import tpu as pltpu
from jax.experimental.pallas import tpu_sc as plsc

_N = 10000
_E = 320000
_D = 128
_DH = _D // 2

_NC = 2
_NS = 16
_NW = _NC * _NS

_CHUNK = 80

_EPW_DEG = _E // _NW
_NCH_DEG = _EPW_DEG // _CHUNK

_EPS = _E // _NS
_NCH = _EPS // _CHUNK

_NPAD = 10000
_RPT = _NPAD // _NS
_ZCH = 125

_BM = 2000
_GRID = _NPAD // _BM

_MESH = plsc.VectorSubcoreMesh(
    core_axis_name="c", subcore_axis_name="s", num_cores=_NC, num_subcores=_NS
)
_SC_PARAMS = pltpu.CompilerParams(use_tc_tiling_on_sc=False)


_DEGW = 16
_DEG_WIN = 8


def _sc_degree_body(dst_hbm, ones_hbm, z16_hbm, deg_hbm, dst_v, ones_v, deg_sh, dsem):
    c = lax.axis_index("c")
    s = lax.axis_index("s")
    wid = s * _NC + c
    pltpu.sync_copy(dst_hbm.at[wid], dst_v)
    pltpu.sync_copy(ones_hbm, ones_v)
    base = s * _RPT
    for k in range(_RPT // _ZCH):
        pltpu.sync_copy(z16_hbm, deg_sh.at[pl.ds(base + k * _ZCH, _ZCH)])
    plsc.subcore_barrier()

    for j in range(_DEG_WIN):
        pltpu.async_copy(ones_v, deg_sh.at[dst_v.at[j]], dsem, add=True)

    def chunk(j, carry):
        pltpu.make_async_copy(ones_v, deg_sh.at[dst_v.at[0]], dsem).wait()
        pltpu.async_copy(ones_v, deg_sh.at[dst_v.at[j + _DEG_WIN]], dsem, add=True)
        return carry

    lax.fori_loop(0, _NCH_DEG - _DEG_WIN, chunk, 0)
    for _ in range(_DEG_WIN):
        pltpu.make_async_copy(ones_v, deg_sh.at[dst_v.at[0]], dsem).wait()
    plsc.subcore_barrier()
    out_base = c * _NPAD + base
    for k in range(_RPT // _ZCH):
        pltpu.sync_copy(
            deg_sh.at[pl.ds(base + k * _ZCH, _ZCH)],
            deg_hbm.at[pl.ds(out_base + k * _ZCH, _ZCH)],
        )


_sc_degree = pl.kernel(
    _sc_degree_body,
    out_type=jax.ShapeDtypeStruct((_NC * _NPAD, _DEGW), jnp.float32),
    mesh=_MESH,
    scratch_types=[
        pltpu.VMEM((_NCH_DEG, _CHUNK), jnp.int32),
        pltpu.VMEM((_CHUNK, _DEGW), jnp.float32),
        pltpu.VMEM_SHARED((_NPAD, _DEGW), jnp.float32),
        pltpu.SemaphoreType.DMA,
    ],
    compiler_params=_SC_PARAMS,
)


_NRING = 5


def _sc_scatter_body(
    g2n_hbm, src2_hbm, dst_hbm, z64_hbm, acc_hbm,
    src_v, dst_v, rbufs, acc_sh, gsems
):
    c = lax.axis_index("c")
    s = lax.axis_index("s")
    pltpu.sync_copy(src2_hbm.at[c * _NS + s], src_v)
    pltpu.sync_copy(dst_hbm.at[s], dst_v)
    base = s * _RPT
    for k in range(_RPT // _ZCH):
        pltpu.sync_copy(z64_hbm, acc_sh.at[pl.ds(base + k * _ZCH, _ZCH)])
    plsc.subcore_barrier()

    for b in range(_NRING):
        pltpu.async_copy(g2n_hbm.at[src_v.at[b]], rbufs[b], gsems[b])

    def outer(o, carry):
        for b in range(_NRING):
            cc = o * _NRING + b
            pltpu.make_async_copy(g2n_hbm.at[src_v.at[cc]], rbufs[b], gsems[b]).wait()
            pltpu.sync_copy(rbufs[b], acc_sh.at[dst_v.at[cc]], add=True)
            nxt = cc + _NRING

            @pl.when(nxt < _NCH)
            def _start_next():
                pltpu.async_copy(g2n_hbm.at[src_v.at[nxt]], rbufs[b], gsems[b])

        return carry

    lax.fori_loop(0, _NCH // _NRING, outer, 0)
    plsc.subcore_barrier()
    out_base = c * _NPAD + base
    for k in range(_RPT // _ZCH):
        pltpu.sync_copy(
            acc_sh.at[pl.ds(base + k * _ZCH, _ZCH)],
            acc_hbm.at[pl.ds(out_base + k * _ZCH, _ZCH)],
        )


_sc_scatter = pl.kernel(
    _sc_scatter_body,
    out_type=jax.ShapeDtypeStruct((_NC * _NPAD, _DH), jnp.float32),
    mesh=_MESH,
    scratch_types=[
        pltpu.VMEM((_NCH, _CHUNK), jnp.int32),
        pltpu.VMEM((_NCH, _CHUNK), jnp.int32),
        tuple(pltpu.VMEM((_CHUNK, _DH), jnp.float32) for _ in range(_NRING)),
        pltpu.VMEM_SHARED((_NPAD, _DH), jnp.float32),
        tuple(pltpu.SemaphoreType.DMA for _ in range(_NRING)),
    ],
    compiler_params=_SC_PARAMS,
)


def _tc_mm_body(x_ref, w_ref, h_ref):
    h_ref[...] = jnp.dot(x_ref[...], w_ref[0], precision=lax.Precision.HIGHEST,
                         preferred_element_type=jnp.float32)


def _tc_a_body(d0_ref, d1_ref, h_ref, g_ref, dinv_ref):
    deg = d0_ref[:, 0:1] + d1_ref[:, 0:1] + 1.0
    dinv = lax.rsqrt(deg)
    g_ref[...] = h_ref[...] * dinv
    dinv_ref[...] = dinv


def _tc_b_body(a0_ref, a1_ref, ge_ref, go_ref, dinv_ref, b_ref, w_ref, g2_ref):
    dinv = dinv_ref[...]
    acc = jnp.concatenate([a0_ref[...], a1_ref[...]], axis=1)
    g = jnp.concatenate([ge_ref[...], go_ref[...]], axis=1)
    z = jnp.maximum(dinv * (acc + g) + b_ref[...], 0.0)
    h2 = jnp.dot(z, w_ref[0], precision=lax.Precision.HIGHEST,
                 preferred_element_type=jnp.float32)
    g2_ref[...] = h2 * dinv


def _tc_c_body(a0_ref, a1_ref, ge_ref, go_ref, dinv_ref, b_ref, o_ref):
    acc = jnp.concatenate([a0_ref[...], a1_ref[...]], axis=1)
    g = jnp.concatenate([ge_ref[...], go_ref[...]], axis=1)
    o_ref[...] = dinv_ref[...] * (acc + g) + b_ref[...]


def _row_half_specs(width):
    return [
        pl.BlockSpec((_BM, width), lambda i: (i, 0)),
        pl.BlockSpec((_BM, width), lambda i: (i + _GRID, 0)),
    ]


def _tc_mm(x, W1s):
    return pl.pallas_call(
        _tc_mm_body,
        grid=(_GRID, _NC),
        in_specs=[
            pl.BlockSpec((_BM, _D), lambda i, j: (i, 0)),
            pl.BlockSpec((1, _D, _DH), lambda i, j: (j, 0, 0)),
        ],
        out_specs=pl.BlockSpec((_BM, _DH), lambda i, j: (i + j * _GRID, 0)),
        out_shape=jax.ShapeDtypeStruct((_NC * _N, _DH), jnp.float32),
    )(x, W1s)


def _tc_a(deg2n, h2n):
    return pl.pallas_call(
        _tc_a_body,
        grid=(_GRID, _NC),
        in_specs=[
            pl.BlockSpec((_BM, _DEGW), lambda i, j: (i, 0)),
            pl.BlockSpec((_BM, _DEGW), lambda i, j: (i + _GRID, 0)),
            pl.BlockSpec((_BM, _DH), lambda i, j: (i + j * _GRID, 0)),
        ],
        out_specs=[
            pl.BlockSpec((_BM, _DH), lambda i, j: (i + j * _GRID, 0)),
            pl.BlockSpec((_BM, 1), lambda i, j: (i, 0)),
        ],
        out_shape=[
            jax.ShapeDtypeStruct((_NC * _N, _DH), jnp.float32),
            jax.ShapeDtypeStruct((_N, 1), jnp.float32),
        ],
    )(deg2n, deg2n, h2n)


def _tc_b(acc2n, g2n, dinv, b1, W2s):
    return pl.pallas_call(
        _tc_b_body,
        grid=(_GRID, _NC),
        in_specs=[
            pl.BlockSpec((_BM, _DH), lambda i, j: (i, 0)),
            pl.BlockSpec((_BM, _DH), lambda i, j: (i + _GRID, 0)),
            pl.BlockSpec((_BM, _DH), lambda i, j: (i, 0)),
            pl.BlockSpec((_BM, _DH), lambda i, j: (i + _GRID, 0)),
            pl.BlockSpec((_BM, 1), lambda i, j: (i, 0)),
            pl.BlockSpec((1, _D), lambda i, j: (0, 0)),
            pl.BlockSpec((1, _D, _DH), lambda i, j: (j, 0, 0)),
        ],
        out_specs=pl.BlockSpec((_BM, _DH), lambda i, j: (i + j * _GRID, 0)),
        out_shape=jax.ShapeDtypeStruct((_NC * _N, _DH), jnp.float32),
    )(acc2n, acc2n, g2n, g2n, dinv, b1, W2s)


def _tc_c(acc2n, g2n, dinv, b2):
    return pl.pallas_call(
        _tc_c_body,
        grid=(_GRID,),
        in_specs=_row_half_specs(_DH) + _row_half_specs(_DH) + [
            pl.BlockSpec((_BM, 1), lambda i: (i, 0)),
            pl.BlockSpec((1, _D), lambda i: (0, 0)),
        ],
        out_specs=pl.BlockSpec((_BM, _D), lambda i: (i, 0)),
        out_shape=jax.ShapeDtypeStruct((_N, _D), jnp.float32),
    )(acc2n, acc2n, g2n, g2n, dinv, b2)


def kernel(x, edge_index, W1, b1, W2, b2):
    ei = edge_index.astype(jnp.int32)
    dst_deg = ei[1].reshape(_NW, _NCH_DEG, _CHUNK)
    src_s = ei[0].reshape(_NS, _NCH, _CHUNK)
    dst_s = ei[1].reshape(_NS, _NCH, _CHUNK)
    src2 = jnp.concatenate([src_s, src_s + _N], axis=0)
    ones16 = jnp.ones((_CHUNK, _DEGW), jnp.float32)
    z16 = jnp.zeros((_ZCH, _DEGW), jnp.float32)
    z64 = jnp.zeros((_ZCH, _DH), jnp.float32)
    b1r = b1.reshape(1, _D)
    b2r = b2.reshape(1, _D)
    W1s = jnp.stack([W1[:, :_DH], W1[:, _DH:]])
    W2s = jnp.stack([W2[:, :_DH], W2[:, _DH:]])

    h1 = _tc_mm(x, W1s)
    deg2n = _sc_degree(dst_deg, ones16, z16)
    g1, dinv = _tc_a(deg2n, h1)
    acc1 = _sc_scatter(g1, src2, dst_s, z64)
    g2 = _tc_b(acc1, g1, dinv, b1r, W2s)
    acc2 = _sc_scatter(g2, src2, dst_s, z64)
    return _tc_c(acc2, g2, dinv, b2r)


# speedup vs baseline: 1.1414x; 1.1414x over previous
import jax
import jax.numpy as jnp
from jax import lax
from jax.experimental import pallas as pl
from jax.experimental.pallas import tpu as pltpu
from jax.experimental.pallas import tpu_sc as plsc

_N = 10000
_E = 320000
_D = 128
_DH = _D // 2

_NC = 2
_NS = 16
_NW = _NC * _NS

_CHUNK = 80

_EPW_DEG = _E // _NW
_NCH_DEG = _EPW_DEG // _CHUNK

_EPS = _E // _NS
_NCH = _EPS // _CHUNK

_NPAD = 10000
_RPT = _NPAD // _NS
_ZCH = 125

_BM = 2000
_GRID = _NPAD // _BM

_MESH = plsc.VectorSubcoreMesh(
    core_axis_name="c", subcore_axis_name="s", num_cores=_NC, num_subcores=_NS
)
_SC_PARAMS = pltpu.CompilerParams(use_tc_tiling_on_sc=False)


_DEGW = 16
_DEG_WIN = 8


def _sc_degree_body(dst_hbm, ones_hbm, z16_hbm, deg_hbm, dst_v, ones_v, deg_sh, dsem):
    c = lax.axis_index("c")
    s = lax.axis_index("s")
    wid = s * _NC + c
    pltpu.sync_copy(dst_hbm.at[wid], dst_v)
    pltpu.sync_copy(ones_hbm, ones_v)
    base = s * _RPT
    for k in range(_RPT // _ZCH):
        pltpu.sync_copy(z16_hbm, deg_sh.at[pl.ds(base + k * _ZCH, _ZCH)])
    plsc.subcore_barrier()

    for j in range(_DEG_WIN):
        pltpu.async_copy(ones_v, deg_sh.at[dst_v.at[j]], dsem, add=True)

    def chunk(j, carry):
        pltpu.make_async_copy(ones_v, deg_sh.at[dst_v.at[0]], dsem).wait()
        pltpu.async_copy(ones_v, deg_sh.at[dst_v.at[j + _DEG_WIN]], dsem, add=True)
        return carry

    lax.fori_loop(0, _NCH_DEG - _DEG_WIN, chunk, 0)
    for _ in range(_DEG_WIN):
        pltpu.make_async_copy(ones_v, deg_sh.at[dst_v.at[0]], dsem).wait()
    plsc.subcore_barrier()
    out_base = c * _NPAD + base
    for k in range(_RPT // _ZCH):
        pltpu.sync_copy(
            deg_sh.at[pl.ds(base + k * _ZCH, _ZCH)],
            deg_hbm.at[pl.ds(out_base + k * _ZCH, _ZCH)],
        )


_sc_degree = pl.kernel(
    _sc_degree_body,
    out_type=jax.ShapeDtypeStruct((_NC * _NPAD, _DEGW), jnp.float32),
    mesh=_MESH,
    scratch_types=[
        pltpu.VMEM((_NCH_DEG, _CHUNK), jnp.int32),
        pltpu.VMEM((_CHUNK, _DEGW), jnp.float32),
        pltpu.VMEM_SHARED((_NPAD, _DEGW), jnp.float32),
        pltpu.SemaphoreType.DMA,
    ],
    compiler_params=_SC_PARAMS,
)


_NRING = 5


def _sc_scatter_body(
    g2n_hbm, src2_hbm, dst_hbm, z64_hbm, acc_hbm,
    src_v, dst_v, rbufs, acc_sh, gsems
):
    c = lax.axis_index("c")
    s = lax.axis_index("s")
    pltpu.sync_copy(src2_hbm.at[c * _NS + s], src_v)
    pltpu.sync_copy(dst_hbm.at[s], dst_v)
    base = s * _RPT
    for k in range(_RPT // _ZCH):
        pltpu.sync_copy(z64_hbm, acc_sh.at[pl.ds(base + k * _ZCH, _ZCH)])
    plsc.subcore_barrier()

    for b in range(_NRING):
        pltpu.async_copy(g2n_hbm.at[src_v.at[b]], rbufs[b], gsems[b])

    def outer(o, carry):
        for b in range(_NRING):
            cc = o * _NRING + b
            pltpu.make_async_copy(g2n_hbm.at[src_v.at[cc]], rbufs[b], gsems[b]).wait()
            pltpu.sync_copy(rbufs[b], acc_sh.at[dst_v.at[cc]], add=True)
            nxt = cc + _NRING

            @pl.when(nxt < _NCH)
            def _start_next():
                pltpu.async_copy(g2n_hbm.at[src_v.at[nxt]], rbufs[b], gsems[b])

        return carry

    lax.fori_loop(0, _NCH // _NRING, outer, 0)
    plsc.subcore_barrier()
    out_base = c * _NPAD + base
    for k in range(_RPT // _ZCH):
        pltpu.sync_copy(
            acc_sh.at[pl.ds(base + k * _ZCH, _ZCH)],
            acc_hbm.at[pl.ds(out_base + k * _ZCH, _ZCH)],
        )


_sc_scatter = pl.kernel(
    _sc_scatter_body,
    out_type=jax.ShapeDtypeStruct((_NC * _NPAD, _DH), jnp.float32),
    mesh=_MESH,
    scratch_types=[
        pltpu.VMEM((_NCH, _CHUNK), jnp.int32),
        pltpu.VMEM((_NCH, _CHUNK), jnp.int32),
        tuple(pltpu.VMEM((_CHUNK, _DH), jnp.float32) for _ in range(_NRING)),
        pltpu.VMEM_SHARED((_NPAD, _DH), jnp.float32),
        tuple(pltpu.SemaphoreType.DMA for _ in range(_NRING)),
    ],
    compiler_params=_SC_PARAMS,
)


def _tc_mm_body(x_ref, w_ref, h_ref):
    h_ref[...] = jnp.dot(x_ref[...], w_ref[...], precision=lax.Precision.HIGHEST,
                         preferred_element_type=jnp.float32)


def _tc_a_body(deg_ref, h_ref, g_ref, dinv_ref):
    dp = deg_ref[...]
    deg = dp[0, :, 0:1] + dp[1, :, 0:1] + 1.0
    dinv = lax.rsqrt(deg)
    g_ref[...] = h_ref[...] * dinv
    dinv_ref[...] = dinv


def _tc_b_body(acc_ref, g_ref, dinv_ref, b_ref, w_ref, g2_ref):
    dinv = dinv_ref[...]
    acc = jnp.concatenate([acc_ref[0], acc_ref[1]], axis=1)
    z = jnp.maximum(dinv * (acc + g_ref[...]) + b_ref[...], 0.0)
    h2 = jnp.dot(z, w_ref[...], precision=lax.Precision.HIGHEST,
                 preferred_element_type=jnp.float32)
    g2_ref[...] = h2 * dinv


def _tc_c_body(acc_ref, g_ref, dinv_ref, b_ref, o_ref):
    acc = jnp.concatenate([acc_ref[0], acc_ref[1]], axis=1)
    o_ref[...] = dinv_ref[...] * (acc + g_ref[...]) + b_ref[...]


def _tc_mm(x, W1):
    return pl.pallas_call(
        _tc_mm_body,
        grid=(_GRID,),
        in_specs=[
            pl.BlockSpec((_BM, _D), lambda i: (i, 0)),
            pl.BlockSpec((_D, _D), lambda i: (0, 0)),
        ],
        out_specs=pl.BlockSpec((_BM, _D), lambda i: (i, 0)),
        out_shape=jax.ShapeDtypeStruct((_N, _D), jnp.float32),
    )(x, W1)


def _tc_a(deg_parts, h1):
    return pl.pallas_call(
        _tc_a_body,
        grid=(_GRID,),
        in_specs=[
            pl.BlockSpec((_NC, _BM, _DEGW), lambda i: (0, i, 0)),
            pl.BlockSpec((_BM, _D), lambda i: (i, 0)),
        ],
        out_specs=[
            pl.BlockSpec((_BM, _D), lambda i: (i, 0)),
            pl.BlockSpec((_BM, 1), lambda i: (i, 0)),
        ],
        out_shape=[
            jax.ShapeDtypeStruct((_N, _D), jnp.float32),
            jax.ShapeDtypeStruct((_N, 1), jnp.float32),
        ],
    )(deg_parts, h1)


def _tc_b(acc_parts, g1, dinv, b1, W2):
    return pl.pallas_call(
        _tc_b_body,
        grid=(_GRID,),
        in_specs=[
            pl.BlockSpec((_NC, _BM, _DH), lambda i: (0, i, 0)),
            pl.BlockSpec((_BM, _D), lambda i: (i, 0)),
            pl.BlockSpec((_BM, 1), lambda i: (i, 0)),
            pl.BlockSpec((1, _D), lambda i: (0, 0)),
            pl.BlockSpec((_D, _D), lambda i: (0, 0)),
        ],
        out_specs=pl.BlockSpec((_BM, _D), lambda i: (i, 0)),
        out_shape=jax.ShapeDtypeStruct((_N, _D), jnp.float32),
    )(acc_parts, g1, dinv, b1, W2)


def _tc_c(acc_parts, g2, dinv, b2):
    return pl.pallas_call(
        _tc_c_body,
        grid=(_GRID,),
        in_specs=[
            pl.BlockSpec((_NC, _BM, _DH), lambda i: (0, i, 0)),
            pl.BlockSpec((_BM, _D), lambda i: (i, 0)),
            pl.BlockSpec((_BM, 1), lambda i: (i, 0)),
            pl.BlockSpec((1, _D), lambda i: (0, 0)),
        ],
        out_specs=pl.BlockSpec((_BM, _D), lambda i: (i, 0)),
        out_shape=jax.ShapeDtypeStruct((_N, _D), jnp.float32),
    )(acc_parts, g2, dinv, b2)


def kernel(x, edge_index, W1, b1, W2, b2):
    ei = edge_index.astype(jnp.int32)
    dst_deg = ei[1].reshape(_NW, _NCH_DEG, _CHUNK)
    src_s = ei[0].reshape(_NS, _NCH, _CHUNK)
    dst_s = ei[1].reshape(_NS, _NCH, _CHUNK)
    src2 = jnp.stack([2 * src_s, 2 * src_s + 1]).reshape(_NC * _NS, _NCH, _CHUNK)
    ones16 = jnp.ones((_CHUNK, _DEGW), jnp.float32)
    z16 = jnp.zeros((_ZCH, _DEGW), jnp.float32)
    z64 = jnp.zeros((_ZCH, _DH), jnp.float32)
    b1r = b1.reshape(1, _D)
    b2r = b2.reshape(1, _D)

    h1 = _tc_mm(x, W1)
    deg_parts = _sc_degree(dst_deg, ones16, z16).reshape(_NC, _NPAD, _DEGW)
    g1, dinv = _tc_a(deg_parts, h1)
    acc1 = _sc_scatter(g1.reshape(2 * _N, _DH), src2, dst_s, z64)
    g2 = _tc_b(acc1.reshape(_NC, _NPAD, _DH), g1, dinv, b1r, W2)
    acc2 = _sc_scatter(g2.reshape(2 * _N, _DH), src2, dst_s, z64)
    return _tc_c(acc2.reshape(_NC, _NPAD, _DH), g2, dinv, b2r)
